# baseline (speedup 1.0000x reference)
import functools

import jax
import jax.numpy as jnp
from jax.experimental import pallas as pl
from jax.experimental.pallas import tpu as pltpu


def _order_body(c_hi, c_lo, C, r_ref, vm_ref, p_ref):
    v = vm_ref[...].reshape(1, C)
    crow = jax.lax.broadcasted_iota(jnp.int32, (C, C), 0)
    clane = jax.lax.broadcasted_iota(jnp.int32, (C, C), 1)
    U = jnp.broadcast_to(v, (C, C))
    vcol = jnp.sum(jnp.where(crow == clane, U, 0.0), axis=1, keepdims=True)
    V = jnp.broadcast_to(vcol, (C, C))
    before = (U > V) | ((U == V) & (clane < crow))
    rank = jnp.sum(before.astype(jnp.int32), axis=1, keepdims=True)

    rv = jnp.full((1, 1), r_ref[0, 0], jnp.float32)
    hi = rv * c_hi
    lo = rv * c_lo
    s = hi + lo
    err = lo - (s - hi)
    n = jnp.floor(s)
    frac = (s - n) + err
    nexf = n + jnp.floor(frac)
    nexi = nexf.astype(jnp.int32)

    p_ref[...] = jnp.where(
        (rank == clane) & (clane < nexi), 1.0, 0.0
    ).reshape(1, C, C)


def _permute_body(JC, nex_ref, x_ref, p_ref, o_ref):
    jc = pl.program_id(2)
    M = x_ref.shape[2] * x_ref.shape[3]
    C = x_ref.shape[4]
    active = jc * JC < nex_ref[0]

    @pl.when(active)
    def _mm():
        xm = x_ref[...].reshape(M, C)
        pm = p_ref[...].reshape(C, JC)
        acc = jax.lax.dot_general(
            xm, pm, (((1,), (0,)), ((), ())),
            precision=jax.lax.Precision.HIGHEST,
            preferred_element_type=jnp.float32,
        )
        o_ref[...] = acc.reshape(o_ref.shape)

    @pl.when(jnp.logical_not(active))
    def _zero():
        o_ref[...] = jnp.zeros_like(o_ref)


def kernel(x, exist_ratio):
    B, C, D, H, W = x.shape
    c_hi = float(1 << (C.bit_length() - 1))
    c_lo = float(C) - c_hi

    val_mean = jnp.mean(jnp.abs(x), axis=(2, 3, 4))

    vm3 = val_mean.reshape(B, 1, C)
    r2 = exist_ratio.reshape(1, 1)

    pmat = pl.pallas_call(
        functools.partial(_order_body, c_hi, c_lo, C),
        grid=(B,),
        in_specs=[
            pl.BlockSpec(memory_space=pltpu.SMEM),
            pl.BlockSpec((1, 1, C), lambda b: (b, 0, 0)),
        ],
        out_specs=pl.BlockSpec((1, C, C), lambda b: (b, 0, 0)),
        out_shape=jax.ShapeDtypeStruct((B, C, C), jnp.float32),
    )(r2, vm3)

    rvs = exist_ratio.astype(jnp.float32)
    hi = rvs * jnp.float32(c_hi)
    lo = rvs * jnp.float32(c_lo)
    s = hi + lo
    err = lo - (s - hi)
    n = jnp.floor(s)
    frac = (s - n) + err
    nexi = (n + jnp.floor(frac)).astype(jnp.int32).reshape(1)

    y = jnp.transpose(x, (0, 2, 3, 4, 1))
    JC = 128
    grid_spec = pltpu.PrefetchScalarGridSpec(
        num_scalar_prefetch=1,
        grid=(B, D, C // JC),
        in_specs=[
            pl.BlockSpec((1, 1, H, W, C), lambda b, d, jc, nn: (b, d, 0, 0, 0)),
            pl.BlockSpec((1, C, JC), lambda b, d, jc, nn: (b, 0, jc)),
        ],
        out_specs=pl.BlockSpec(
            (1, 1, H, W, JC), lambda b, d, jc, nn: (b, d, 0, 0, jc)
        ),
    )
    out_perm = pl.pallas_call(
        functools.partial(_permute_body, JC),
        grid_spec=grid_spec,
        out_shape=jax.ShapeDtypeStruct((B, D, H, W, C), x.dtype),
    )(nexi, y, pmat)
    return jnp.transpose(out_perm, (0, 4, 1, 2, 3))


# speedup vs baseline: 1.3895x; 1.3895x over previous
import functools

import jax
import jax.numpy as jnp
from jax.experimental import pallas as pl
from jax.experimental.pallas import tpu as pltpu


def _order_body(c_hi, c_lo, C, r_ref, vm_ref, gidx_ref, nex_ref):
    v = vm_ref[...].reshape(1, C)
    crow = jax.lax.broadcasted_iota(jnp.int32, (C, C), 0)
    clane = jax.lax.broadcasted_iota(jnp.int32, (C, C), 1)
    U = jnp.broadcast_to(v, (C, C))
    vcol = jnp.sum(jnp.where(crow == clane, U, 0.0), axis=1, keepdims=True)
    V = jnp.broadcast_to(vcol, (C, C))
    before = (U > V) | ((U == V) & (clane < crow))
    rank = jnp.sum(before.astype(jnp.int32), axis=1, keepdims=True)
    wmat = jnp.where(rank == clane, crow, 0)
    order = jnp.sum(wmat, axis=0, keepdims=True)

    rv = jnp.full((1, 1), r_ref[0, 0], jnp.float32)
    hi = rv * c_hi
    lo = rv * c_lo
    s = hi + lo
    err = lo - (s - hi)
    n = jnp.floor(s)
    frac = (s - n) + err
    nexf = n + jnp.floor(frac)
    nexi = nexf.astype(jnp.int32)

    gidx_ref[...] = order.reshape(1, 1, C)
    nex_ref[...] = jnp.broadcast_to(nexi, (1, 1, 128))


def _permute_body(JC, nex_ref, x_ref, g_ref, o_ref):
    jc = pl.program_id(2)
    M = x_ref.shape[2] * x_ref.shape[3]
    C = x_ref.shape[4]
    nex = nex_ref[0]
    active = jc * JC < nex

    @pl.when(active)
    def _gather():
        xm = x_ref[...].reshape(M, C)
        idxt = jnp.broadcast_to(g_ref[...].reshape(1, JC), (M, JC))
        local = jnp.bitwise_and(idxt, JC - 1)
        tile = jnp.right_shift(idxt, 7)
        acc = jnp.zeros((M, JC), jnp.float32)
        for s_tile in range(C // JC):
            xs = xm[:, s_tile * JC:(s_tile + 1) * JC]
            gs = jnp.take_along_axis(xs, local, axis=1)
            acc = jnp.where(tile == s_tile, gs, acc)
        jvec = jax.lax.broadcasted_iota(jnp.int32, (M, JC), 1) + jc * JC
        o_ref[...] = jnp.where(jvec < nex, acc, 0.0).reshape(o_ref.shape)

    @pl.when(jnp.logical_not(active))
    def _zero():
        o_ref[...] = jnp.zeros_like(o_ref)


def kernel(x, exist_ratio):
    B, C, D, H, W = x.shape
    c_hi = float(1 << (C.bit_length() - 1))
    c_lo = float(C) - c_hi

    val_mean = jnp.mean(jnp.abs(x), axis=(2, 3, 4))
    vm3 = val_mean.reshape(B, 1, C)
    r2 = exist_ratio.reshape(1, 1)

    gidx3, nexv = pl.pallas_call(
        functools.partial(_order_body, c_hi, c_lo, C),
        grid=(B,),
        in_specs=[
            pl.BlockSpec(memory_space=pltpu.SMEM),
            pl.BlockSpec((1, 1, C), lambda b: (b, 0, 0)),
        ],
        out_specs=[
            pl.BlockSpec((1, 1, C), lambda b: (b, 0, 0)),
            pl.BlockSpec((1, 1, 128), lambda b: (0, 0, 0)),
        ],
        out_shape=[
            jax.ShapeDtypeStruct((B, 1, C), jnp.int32),
            jax.ShapeDtypeStruct((1, 1, 128), jnp.int32),
        ],
    )(r2, vm3)

    nex1 = nexv.reshape(128)[:1]
    y = jnp.transpose(x, (0, 2, 3, 4, 1))
    JC = 128
    grid_spec = pltpu.PrefetchScalarGridSpec(
        num_scalar_prefetch=1,
        grid=(B, D, C // JC),
        in_specs=[
            pl.BlockSpec((1, 1, H, W, C), lambda b, d, jc, nn: (b, d, 0, 0, 0)),
            pl.BlockSpec((1, 1, JC), lambda b, d, jc, nn: (b, 0, jc)),
        ],
        out_specs=pl.BlockSpec(
            (1, 1, H, W, JC), lambda b, d, jc, nn: (b, d, 0, 0, jc)
        ),
    )
    out_perm = pl.pallas_call(
        functools.partial(_permute_body, JC),
        grid_spec=grid_spec,
        out_shape=jax.ShapeDtypeStruct((B, D, H, W, C), x.dtype),
    )(nex1, y, gidx3)
    return jnp.transpose(out_perm, (0, 4, 1, 2, 3))


# speedup vs baseline: 1.5596x; 1.1224x over previous
import functools

import jax
import jax.numpy as jnp
from jax.experimental import pallas as pl
from jax.experimental.pallas import tpu as pltpu


def _order_body(c_hi, c_lo, C, r_ref, vm_ref, gidx_ref, nex_ref):
    v = vm_ref[...].reshape(1, C)
    crow = jax.lax.broadcasted_iota(jnp.int32, (C, C), 0)
    clane = jax.lax.broadcasted_iota(jnp.int32, (C, C), 1)
    U = jnp.broadcast_to(v, (C, C))
    vcol = jnp.sum(jnp.where(crow == clane, U, 0.0), axis=1, keepdims=True)
    V = jnp.broadcast_to(vcol, (C, C))
    before = (U > V) | ((U == V) & (clane < crow))
    rank = jnp.sum(before.astype(jnp.int32), axis=1, keepdims=True)
    wmat = jnp.where(rank == clane, crow, 0)
    order = jnp.sum(wmat, axis=0, keepdims=True)

    rv = jnp.full((1, 1), r_ref[0, 0], jnp.float32)
    hi = rv * c_hi
    lo = rv * c_lo
    s = hi + lo
    err = lo - (s - hi)
    n = jnp.floor(s)
    frac = (s - n) + err
    nexf = n + jnp.floor(frac)
    nexi = nexf.astype(jnp.int32)

    gidx_ref[...] = order.reshape(1, 1, C)
    nex_ref[...] = jnp.broadcast_to(nexi, (1, 1, 128))


def _permute_body(JC, nex_ref, x_ref, g_ref, o_ref):
    jc = pl.program_id(2)
    M = x_ref.shape[1] * x_ref.shape[2] * x_ref.shape[3]
    C = x_ref.shape[4]
    nex = nex_ref[0]
    active = jc * JC < nex

    @pl.when(active)
    def _gather():
        xm = x_ref[...].reshape(M, C)
        idxt = jnp.broadcast_to(g_ref[...].reshape(1, JC), (M, JC))
        local = jnp.bitwise_and(idxt, JC - 1)
        tile = jnp.right_shift(idxt, 7)
        acc = jnp.zeros((M, JC), jnp.float32)
        for s_tile in range(C // JC):
            xs = xm[:, s_tile * JC:(s_tile + 1) * JC]
            gs = jnp.take_along_axis(xs, local, axis=1)
            acc = jnp.where(tile == s_tile, gs, acc)
        jvec = jax.lax.broadcasted_iota(jnp.int32, (M, JC), 1) + jc * JC
        o_ref[...] = jnp.where(jvec < nex, acc, 0.0).reshape(o_ref.shape)

    @pl.when(jnp.logical_not(active))
    def _zero():
        o_ref[...] = jnp.zeros_like(o_ref)


def kernel(x, exist_ratio):
    B, C, D, H, W = x.shape
    c_hi = float(1 << (C.bit_length() - 1))
    c_lo = float(C) - c_hi

    val_mean = jnp.mean(jnp.abs(x), axis=(2, 3, 4))
    vm3 = val_mean.reshape(B, 1, C)
    r2 = exist_ratio.reshape(1, 1)

    gidx3, nexv = pl.pallas_call(
        functools.partial(_order_body, c_hi, c_lo, C),
        grid=(B,),
        in_specs=[
            pl.BlockSpec(memory_space=pltpu.SMEM),
            pl.BlockSpec((1, 1, C), lambda b: (b, 0, 0)),
        ],
        out_specs=[
            pl.BlockSpec((1, 1, C), lambda b: (b, 0, 0)),
            pl.BlockSpec((1, 1, 128), lambda b: (0, 0, 0)),
        ],
        out_shape=[
            jax.ShapeDtypeStruct((B, 1, C), jnp.int32),
            jax.ShapeDtypeStruct((1, 1, 128), jnp.int32),
        ],
    )(r2, vm3)

    nex1 = nexv.reshape(128)[:1]
    y = jnp.transpose(x, (0, 2, 3, 4, 1))
    JC = 128
    DB = 2
    grid_spec = pltpu.PrefetchScalarGridSpec(
        num_scalar_prefetch=1,
        grid=(B, D // DB, C // JC),
        in_specs=[
            pl.BlockSpec((1, DB, H, W, C), lambda b, d, jc, nn: (b, d, 0, 0, 0)),
            pl.BlockSpec((1, 1, JC), lambda b, d, jc, nn: (b, 0, jc)),
        ],
        out_specs=pl.BlockSpec(
            (1, DB, H, W, JC), lambda b, d, jc, nn: (b, d, 0, 0, jc)
        ),
    )
    out_perm = pl.pallas_call(
        functools.partial(_permute_body, JC),
        grid_spec=grid_spec,
        out_shape=jax.ShapeDtypeStruct((B, D, H, W, C), x.dtype),
    )(nex1, y, gidx3)
    return jnp.transpose(out_perm, (0, 4, 1, 2, 3))


# speedup vs baseline: 2.1527x; 1.3803x over previous
import functools

import jax
import jax.numpy as jnp
from jax.experimental import pallas as pl
from jax.experimental.pallas import tpu as pltpu


def _order_body(c_hi, c_lo, C, r_ref, vm_ref, gidx_ref, nex_ref):
    v = vm_ref[...].reshape(1, C)
    crow = jax.lax.broadcasted_iota(jnp.int32, (C, C), 0)
    clane = jax.lax.broadcasted_iota(jnp.int32, (C, C), 1)
    U = jnp.broadcast_to(v, (C, C))
    vcol = jnp.sum(jnp.where(crow == clane, U, 0.0), axis=1, keepdims=True)
    V = jnp.broadcast_to(vcol, (C, C))
    before = (U > V) | ((U == V) & (clane < crow))
    rank = jnp.sum(before.astype(jnp.int32), axis=1, keepdims=True)
    wmat = jnp.where(rank == clane, crow, 0)
    order = jnp.sum(wmat, axis=0, keepdims=True)

    rv = jnp.full((1, 1), r_ref[0, 0], jnp.float32)
    hi = rv * c_hi
    lo = rv * c_lo
    s = hi + lo
    err = lo - (s - hi)
    n = jnp.floor(s)
    frac = (s - n) + err
    nexf = n + jnp.floor(frac)
    nexi = nexf.astype(jnp.int32)

    gidx_ref[...] = order.reshape(1, 1, C)
    nex_ref[...] = jnp.broadcast_to(nexi, (1, 1, 128))


def _permute_body(JC, nex_ref, x_ref, g_ref, o_ref):
    M = x_ref.shape[1] * x_ref.shape[2] * x_ref.shape[3]
    C = x_ref.shape[4]
    nex = nex_ref[0]
    xm = x_ref[...].reshape(M, C)
    NT = C // JC
    for t in range(NT):
        active = t * JC < nex

        @pl.when(active)
        def _gather(t=t):
            idxt = jnp.broadcast_to(
                g_ref[0, 0, t * JC:(t + 1) * JC].reshape(1, JC), (M, JC)
            )
            local = jnp.bitwise_and(idxt, JC - 1)
            tile = jnp.right_shift(idxt, 7)
            acc = jnp.zeros((M, JC), jnp.float32)
            for s_tile in range(NT):
                xs = xm[:, s_tile * JC:(s_tile + 1) * JC]
                gs = jnp.take_along_axis(xs, local, axis=1)
                acc = jnp.where(tile == s_tile, gs, acc)
            jvec = jax.lax.broadcasted_iota(jnp.int32, (M, JC), 1) + t * JC
            o_ref[..., t * JC:(t + 1) * JC] = jnp.where(
                jvec < nex, acc, 0.0
            ).reshape(o_ref.shape[:-1] + (JC,))

        @pl.when(jnp.logical_not(active))
        def _zero(t=t):
            o_ref[..., t * JC:(t + 1) * JC] = jnp.zeros(
                o_ref.shape[:-1] + (JC,), o_ref.dtype
            )


def kernel(x, exist_ratio):
    B, C, D, H, W = x.shape
    c_hi = float(1 << (C.bit_length() - 1))
    c_lo = float(C) - c_hi

    val_mean = jnp.mean(jnp.abs(x), axis=(2, 3, 4))
    vm3 = val_mean.reshape(B, 1, C)
    r2 = exist_ratio.reshape(1, 1)

    gidx3, nexv = pl.pallas_call(
        functools.partial(_order_body, c_hi, c_lo, C),
        grid=(B,),
        in_specs=[
            pl.BlockSpec(memory_space=pltpu.SMEM),
            pl.BlockSpec((1, 1, C), lambda b: (b, 0, 0)),
        ],
        out_specs=[
            pl.BlockSpec((1, 1, C), lambda b: (b, 0, 0)),
            pl.BlockSpec((1, 1, 128), lambda b: (0, 0, 0)),
        ],
        out_shape=[
            jax.ShapeDtypeStruct((B, 1, C), jnp.int32),
            jax.ShapeDtypeStruct((1, 1, 128), jnp.int32),
        ],
    )(r2, vm3)

    nex1 = nexv.reshape(128)[:1]
    y = jnp.transpose(x, (0, 2, 3, 4, 1))
    JC = 128
    DB = 2
    grid_spec = pltpu.PrefetchScalarGridSpec(
        num_scalar_prefetch=1,
        grid=(B, D // DB),
        in_specs=[
            pl.BlockSpec((1, DB, H, W, C), lambda b, d, nn: (b, d, 0, 0, 0)),
            pl.BlockSpec((1, 1, C), lambda b, d, nn: (b, 0, 0)),
        ],
        out_specs=pl.BlockSpec(
            (1, DB, H, W, C), lambda b, d, nn: (b, d, 0, 0, 0)
        ),
    )
    out_perm = pl.pallas_call(
        functools.partial(_permute_body, JC),
        grid_spec=grid_spec,
        out_shape=jax.ShapeDtypeStruct((B, D, H, W, C), x.dtype),
    )(nex1, y, gidx3)
    return jnp.transpose(out_perm, (0, 4, 1, 2, 3))
